# baseline (speedup 1.0000x reference)
import dataclasses
import functools

import jax
import jax.numpy as jnp
from jax import lax
from jax.experimental import pallas as pl
from jax.experimental.pallas import tpu as pltpu
from jax.experimental.pallas import tpu_sc as plsc

_B = 4096
_N = 204800
_V = 100000
_D = 512
_H = 256

_NW = 32
_VPW = 3200
_VPAD = _NW * _VPW
_LANES = 16
_IW = 5120
_NWIN = _N // _IW
_SROWS = _B // _NW

_KB = 10000
_NKB = _V // _KB

_RB = 512
_NRB = _B // _RB


def _sc_emb_and_counts(indices, table):
  mesh = plsc.VectorSubcoreMesh(core_axis_name="c", subcore_axis_name="s")
  cp = pltpu.CompilerParams()
  if "needs_layout_passes" in pltpu.CompilerParams.__dataclass_fields__:
    cp = dataclasses.replace(cp, needs_layout_passes=False)

  @functools.partial(
      pl.kernel,
      compiler_params=cp,
      out_type=[
          jax.ShapeDtypeStruct((_B, _D), jnp.float32),
          jax.ShapeDtypeStruct((_V,), jnp.float32),
      ],
      mesh=mesh,
      scratch_types=[
          pltpu.VMEM((_LANES * _VPW,), jnp.float32),
          pltpu.VMEM((_IW,), jnp.int32),
          pltpu.VMEM((_IW,), jnp.int32),
          pltpu.VMEM((_SROWS,), jnp.int32),
          pltpu.VMEM((_SROWS, _D), jnp.float32),
          pltpu.SemaphoreType.DMA,
          pltpu.SemaphoreType.DMA,
          pltpu.SemaphoreType.DMA,
      ],
  )
  def k(idx_hbm, tbl_hbm, emb_hbm, cnt_hbm, hist_v, iwin0_v, iwin1_v, sidx_v,
        rows_v, gsem, wsem0, wsem1):
    wid = lax.axis_index("s") * 2 + lax.axis_index("c")

    sbase = wid * _SROWS
    pltpu.sync_copy(idx_hbm.at[pl.ds(sbase, _SROWS)], sidx_v)
    gather = pltpu.async_copy(tbl_hbm.at[sidx_v], rows_v, gsem)

    @plsc.parallel_loop(0, _LANES * _VPW, step=_LANES, unroll=8)
    def _(i):
      hist_v[pl.ds(i, _LANES)] = jnp.zeros((_LANES,), jnp.float32)

    lo = wid * _VPW
    laneoff = lax.iota(jnp.int32, _LANES) * _VPW
    ones = jnp.ones((_LANES,), jnp.float32)

    def hist_window(buf):
      @plsc.parallel_loop(0, _IW, step=_LANES, unroll=16)
      def _(j):
        v = buf[pl.ds(j, _LANES)]
        local = v - lo
        m = plsc.bitcast(local, jnp.uint32) < jnp.uint32(_VPW)
        plsc.addupdate_scatter(hist_v, [laneoff + local], ones, mask=m)

    pltpu.async_copy(idx_hbm.at[pl.ds(0, _IW)], iwin0_v, wsem0)

    @pl.loop(0, _NWIN, step=2)
    def _(w):
      pltpu.async_copy(idx_hbm.at[pl.ds((w + 1) * _IW, _IW)], iwin1_v, wsem1)
      pltpu.make_async_copy(idx_hbm.at[pl.ds(w * _IW, _IW)], iwin0_v,
                            wsem0).wait()
      hist_window(iwin0_v)

      @pl.when(w + 2 < _NWIN)
      def _():
        pltpu.async_copy(idx_hbm.at[pl.ds((w + 2) * _IW, _IW)], iwin0_v,
                         wsem0)

      pltpu.make_async_copy(idx_hbm.at[pl.ds((w + 1) * _IW, _IW)], iwin1_v,
                            wsem1).wait()
      hist_window(iwin1_v)

    gather.wait()
    pltpu.sync_copy(rows_v, emb_hbm.at[pl.ds(sbase, _SROWS)])

    @plsc.parallel_loop(0, _VPW, step=_LANES, unroll=2)
    def _(j):
      acc = hist_v[pl.ds(j, _LANES)]
      for l in range(1, _LANES):
        acc = acc + hist_v[pl.ds(l * _VPW + j, _LANES)]
      hist_v[pl.ds(j, _LANES)] = acc

    tail = _V - (_NW - 1) * _VPW

    @pl.when(wid < _NW - 1)
    def _():
      pltpu.sync_copy(hist_v.at[pl.ds(0, _VPW)], cnt_hbm.at[pl.ds(lo, _VPW)])

    @pl.when(wid == _NW - 1)
    def _():
      pltpu.sync_copy(hist_v.at[pl.ds(0, tail)],
                      cnt_hbm.at[pl.ds((_NW - 1) * _VPW, tail)])

  return k(indices, table)


def _tc_fused(counts, table, emb, ln_g, ln_b, fc_w, fc_b, pp_w, pp_b, op_w,
              op_b, tg_w, tg_b, bn_w, bn_b, vl_w, vl_b):
  c3 = counts.reshape(_NKB, 1, _KB)

  def f(c_ref, t_ref, emb_ref, g_ref, b_ref, fcw_ref, fcb_ref, ppw_ref,
        ppb_ref, opw_ref, opb_ref, tgw_ref, tgb_ref, bnw_ref, bnb_ref,
        vlw_ref, vlb_ref, pp_ref, op_ref, tg_ref, bn_ref, vl_ref, sum_ref,
        acc_ref):
    i = pl.program_id(0)

    @pl.when(i == 0)
    def _():
      sum_ref[...] = jnp.zeros_like(sum_ref)
      acc_ref[...] = jnp.zeros_like(acc_ref)

    @pl.when(i < _NKB)
    def _():
      cb = jnp.broadcast_to(c_ref[0], (8, _KB))
      sum_ref[...] += jnp.dot(cb, t_ref[...],
                              preferred_element_type=jnp.float32)

    @pl.when(i >= _NKB)
    def _():
      x = emb_ref[...]
      acc_ref[0:1, :] += jnp.sum(x, axis=0, keepdims=True)

      colsum = acc_ref[0:1, :]
      row_last = sum_ref[0:1, :] - (colsum - x[_RB - 1:_RB, :])
      rows = lax.broadcasted_iota(jnp.int32, (_RB, 1), 0)
      replace = (i == _NKB + _NRB - 1) & (rows == _RB - 1)
      x = jnp.where(replace, row_last, x)

      mean = jnp.mean(x, axis=1, keepdims=True)
      xc = x - mean
      var = jnp.mean(xc * xc, axis=1, keepdims=True)
      xn = xc * lax.rsqrt(var + 1e-5) * g_ref[...] + b_ref[...]

      def dot_nt(a, w):
        return lax.dot_general(a, w, (((1,), (1,)), ((), ())),
                               preferred_element_type=jnp.float32)

      h = jnp.maximum(dot_nt(xn, fcw_ref[...]) + fcb_ref[...], 0.0)
      pp_ref[...] = dot_nt(h, ppw_ref[...]) + ppb_ref[...]
      op_ref[...] = dot_nt(h, opw_ref[...]) + opb_ref[...]
      tg_ref[...] = dot_nt(h, tgw_ref[...]) + tgb_ref[...]
      bn_ref[...] = dot_nt(h, bnw_ref[...]) + bnb_ref[...]
      vl_ref[...] = jnp.tanh(
          jnp.sum(h * vlw_ref[...], axis=1, keepdims=True) + vlb_ref[0, 0])

  full = lambda shape: pl.BlockSpec(shape, lambda i: tuple(0 for _ in shape))
  mvi = lambda i: jnp.minimum(i, _NKB - 1)
  mbi = lambda i: jnp.clip(i - _NKB, 0, _NRB - 1)
  return pl.pallas_call(
      f,
      grid=(_NKB + _NRB,),
      in_specs=[
          pl.BlockSpec((1, 1, _KB), lambda i: (mvi(i), 0, 0)),
          pl.BlockSpec((_KB, _D), lambda i: (mvi(i), 0)),
          pl.BlockSpec((_RB, _D), lambda i: (mbi(i), 0)),
          full((1, _D)), full((1, _D)),
          full((_H, _D)), full((1, _H)),
          full((_D, _H)), full((1, _D)),
          full((_D, _H)), full((1, _D)),
          full((_D, _H)), full((1, _D)),
          full((2, _H)), full((1, 2)),
          full((1, _H)), full((1, 1)),
      ],
      out_specs=[
          pl.BlockSpec((_RB, _D), lambda i: (mbi(i), 0)),
          pl.BlockSpec((_RB, _D), lambda i: (mbi(i), 0)),
          pl.BlockSpec((_RB, _D), lambda i: (mbi(i), 0)),
          pl.BlockSpec((_RB, 2), lambda i: (mbi(i), 0)),
          pl.BlockSpec((_RB, 1), lambda i: (mbi(i), 0)),
      ],
      out_shape=[
          jax.ShapeDtypeStruct((_B, _D), jnp.float32),
          jax.ShapeDtypeStruct((_B, _D), jnp.float32),
          jax.ShapeDtypeStruct((_B, _D), jnp.float32),
          jax.ShapeDtypeStruct((_B, 2), jnp.float32),
          jax.ShapeDtypeStruct((_B, 1), jnp.float32),
      ],
      scratch_shapes=[pltpu.VMEM((8, _D), jnp.float32),
                      pltpu.VMEM((8, _D), jnp.float32)],
  )(c3, table, emb, ln_g.reshape(1, _D), ln_b.reshape(1, _D), fc_w,
    fc_b.reshape(1, _H), pp_w, pp_b.reshape(1, _D), op_w, op_b.reshape(1, _D),
    tg_w, tg_b.reshape(1, _D), bn_w, bn_b.reshape(1, 2), vl_w,
    vl_b.reshape(1, 1))


def kernel(indices, offsets, table, ln_g, ln_b, fc_w, fc_b, pp_w, pp_b,
           op_w, op_b, tg_w, tg_b, bn_w, bn_b, vl_w, vl_b):
  del offsets
  emb, counts = _sc_emb_and_counts(indices, table)
  pp, op, tg, bn, vl = _tc_fused(counts, table, emb, ln_g, ln_b, fc_w, fc_b,
                                 pp_w, pp_b, op_w, op_b, tg_w, tg_b, bn_w,
                                 bn_b, vl_w, vl_b)
  return (pp, op, tg, bn, vl[:, 0])


# speedup vs baseline: 1.3094x; 1.3094x over previous
import dataclasses
import functools

import jax
import jax.numpy as jnp
from jax import lax
from jax.experimental import pallas as pl
from jax.experimental.pallas import tpu as pltpu
from jax.experimental.pallas import tpu_sc as plsc

_B = 4096
_N = 204800
_V = 100000
_D = 512
_H = 256

_NW = 32
_NS = 16
_LANES = 16
_IPT = _N // _NW
_SROWS = _B // _NW

_KB = 5000
_NKB = _V // _KB

_RB = 512
_NRB = _B // _RB


def _sc_emb_and_counts(indices, table):
  mesh = plsc.VectorSubcoreMesh(core_axis_name="c", subcore_axis_name="s")
  cp = pltpu.CompilerParams()
  if "needs_layout_passes" in pltpu.CompilerParams.__dataclass_fields__:
    cp = dataclasses.replace(cp, needs_layout_passes=False)

  @functools.partial(
      pl.kernel,
      compiler_params=cp,
      out_type=[
          jax.ShapeDtypeStruct((_B, _D), jnp.float32),
          jax.ShapeDtypeStruct((2 * _V,), jnp.float32),
      ],
      mesh=mesh,
      scratch_types=[
          pltpu.VMEM_SHARED((_NS * _IPT,), jnp.float32),
          pltpu.VMEM((_IPT,), jnp.float32),
          pltpu.VMEM((_IPT,), jnp.float32),
          pltpu.VMEM((_IPT,), jnp.int32),
          pltpu.VMEM((_SROWS,), jnp.int32),
          pltpu.VMEM((_SROWS, _D), jnp.float32),
          pltpu.SemaphoreType.DMA,
          pltpu.SemaphoreType.DMA,
      ],
  )
  def k(idx_hbm, tbl_hbm, emb_hbm, cnt_hbm, hist_s, zv, ones_v, myidx_v,
        sidx_v, rows_v, gsem, isem):
    cid = lax.axis_index("c")
    sid = lax.axis_index("s")
    wid = sid * 2 + cid

    sbase = wid * _SROWS
    pltpu.sync_copy(idx_hbm.at[pl.ds(sbase, _SROWS)], sidx_v)
    gather = pltpu.async_copy(tbl_hbm.at[sidx_v], rows_v, gsem)
    idxcp = pltpu.async_copy(idx_hbm.at[pl.ds(wid * _IPT, _IPT)], myidx_v,
                             isem)

    @plsc.parallel_loop(0, _IPT, step=_LANES, unroll=8)
    def _(i):
      zv[pl.ds(i, _LANES)] = jnp.zeros((_LANES,), jnp.float32)
      ones_v[pl.ds(i, _LANES)] = jnp.ones((_LANES,), jnp.float32)

    pltpu.sync_copy(zv, hist_s.at[pl.ds(sid * _IPT, _IPT)])
    plsc.subcore_barrier()

    idxcp.wait()
    pltpu.sync_copy(ones_v, hist_s.at[myidx_v], add=True)
    plsc.subcore_barrier()

    tail = _V - (_NS - 1) * _IPT

    pltpu.sync_copy(hist_s.at[pl.ds(sid * _IPT, _IPT)], zv)

    @pl.when(sid < _NS - 1)
    def _():
      pltpu.sync_copy(zv, cnt_hbm.at[pl.ds(cid * _V + sid * _IPT, _IPT)])

    @pl.when(sid == _NS - 1)
    def _():
      pltpu.sync_copy(zv.at[pl.ds(0, tail)],
                      cnt_hbm.at[pl.ds(cid * _V + (_NS - 1) * _IPT, tail)])

    gather.wait()
    pltpu.sync_copy(rows_v, emb_hbm.at[pl.ds(sbase, _SROWS)])

  return k(indices, table)


def _tc_fused(counts, table, emb, ln_g, ln_b, fc_w, fc_b, pp_w, pp_b, op_w,
              op_b, tg_w, tg_b, bn_w, bn_b, vl_w, vl_b):
  c4 = counts.reshape(2, _NKB, 1, _KB)

  def f(c_ref, t_ref, emb_ref, g_ref, b_ref, fcw_ref, fcb_ref, ppw_ref,
        ppb_ref, opw_ref, opb_ref, tgw_ref, tgb_ref, bnw_ref, bnb_ref,
        vlw_ref, vlb_ref, pp_ref, op_ref, tg_ref, bn_ref, vl_ref, sum_ref,
        acc_ref):
    i = pl.program_id(0)

    @pl.when(i == 0)
    def _():
      sum_ref[...] = jnp.zeros_like(sum_ref)
      acc_ref[...] = jnp.zeros_like(acc_ref)

    @pl.when(i < _NKB)
    def _():
      c = c_ref[0, 0] + c_ref[1, 0]
      cb = jnp.broadcast_to(c, (8, _KB))
      sum_ref[...] += jnp.dot(cb, t_ref[...],
                              preferred_element_type=jnp.float32)

    @pl.when(i >= _NKB)
    def _():
      x = emb_ref[...]
      acc_ref[0:1, :] += jnp.sum(x, axis=0, keepdims=True)

      colsum = acc_ref[0:1, :]
      row_last = sum_ref[0:1, :] - (colsum - x[_RB - 1:_RB, :])
      rows = lax.broadcasted_iota(jnp.int32, (_RB, 1), 0)
      replace = (i == _NKB + _NRB - 1) & (rows == _RB - 1)
      x = jnp.where(replace, row_last, x)

      mean = jnp.mean(x, axis=1, keepdims=True)
      xc = x - mean
      var = jnp.mean(xc * xc, axis=1, keepdims=True)
      xn = xc * lax.rsqrt(var + 1e-5) * g_ref[...] + b_ref[...]

      def dot_nt(a, w):
        return lax.dot_general(a, w, (((1,), (1,)), ((), ())),
                               preferred_element_type=jnp.float32)

      h = jnp.maximum(dot_nt(xn, fcw_ref[...]) + fcb_ref[...], 0.0)
      pp_ref[...] = dot_nt(h, ppw_ref[...]) + ppb_ref[...]
      op_ref[...] = dot_nt(h, opw_ref[...]) + opb_ref[...]
      tg_ref[...] = dot_nt(h, tgw_ref[...]) + tgb_ref[...]
      bn_ref[...] = dot_nt(h, bnw_ref[...]) + bnb_ref[...]
      vl_ref[...] = jnp.tanh(
          jnp.sum(h * vlw_ref[...], axis=1, keepdims=True) + vlb_ref[0, 0])

  full = lambda shape: pl.BlockSpec(shape, lambda i: tuple(0 for _ in shape))
  mvi = lambda i: jnp.minimum(i, _NKB - 1)
  mbi = lambda i: jnp.clip(i - _NKB, 0, _NRB - 1)
  return pl.pallas_call(
      f,
      grid=(_NKB + _NRB,),
      in_specs=[
          pl.BlockSpec((2, 1, 1, _KB), lambda i: (0, mvi(i), 0, 0)),
          pl.BlockSpec((_KB, _D), lambda i: (mvi(i), 0)),
          pl.BlockSpec((_RB, _D), lambda i: (mbi(i), 0)),
          full((1, _D)), full((1, _D)),
          full((_H, _D)), full((1, _H)),
          full((_D, _H)), full((1, _D)),
          full((_D, _H)), full((1, _D)),
          full((_D, _H)), full((1, _D)),
          full((2, _H)), full((1, 2)),
          full((1, _H)), full((1, 1)),
      ],
      out_specs=[
          pl.BlockSpec((_RB, _D), lambda i: (mbi(i), 0)),
          pl.BlockSpec((_RB, _D), lambda i: (mbi(i), 0)),
          pl.BlockSpec((_RB, _D), lambda i: (mbi(i), 0)),
          pl.BlockSpec((_RB, 2), lambda i: (mbi(i), 0)),
          pl.BlockSpec((_RB, 1), lambda i: (mbi(i), 0)),
      ],
      out_shape=[
          jax.ShapeDtypeStruct((_B, _D), jnp.float32),
          jax.ShapeDtypeStruct((_B, _D), jnp.float32),
          jax.ShapeDtypeStruct((_B, _D), jnp.float32),
          jax.ShapeDtypeStruct((_B, 2), jnp.float32),
          jax.ShapeDtypeStruct((_B, 1), jnp.float32),
      ],
      scratch_shapes=[pltpu.VMEM((8, _D), jnp.float32),
                      pltpu.VMEM((8, _D), jnp.float32)],
  )(c4, table, emb, ln_g.reshape(1, _D), ln_b.reshape(1, _D), fc_w,
    fc_b.reshape(1, _H), pp_w, pp_b.reshape(1, _D), op_w, op_b.reshape(1, _D),
    tg_w, tg_b.reshape(1, _D), bn_w, bn_b.reshape(1, 2), vl_w,
    vl_b.reshape(1, 1))


def kernel(indices, offsets, table, ln_g, ln_b, fc_w, fc_b, pp_w, pp_b,
           op_w, op_b, tg_w, tg_b, bn_w, bn_b, vl_w, vl_b):
  del offsets
  emb, counts = _sc_emb_and_counts(indices, table)
  pp, op, tg, bn, vl = _tc_fused(counts, table, emb, ln_g, ln_b, fc_w, fc_b,
                                 pp_w, pp_b, op_w, op_b, tg_w, tg_b, bn_w,
                                 bn_b, vl_w, vl_b)
  return (pp, op, tg, bn, vl[:, 0])
